# Initial kernel scaffold; baseline (speedup 1.0000x reference)
import jax
import jax.numpy as jnp
from jax.experimental import pallas as pl


def kernel(x, Wr, br, W1, b1, W2, b2):
    raise NotImplementedError("write your pallas kernel here")


# speedup vs baseline: 3.2146x; 3.2146x over previous
import jax
import jax.numpy as jnp
from jax import lax
from jax.experimental import pallas as pl
from jax.experimental.pallas import tpu as pltpu

NUM_EXPERTS = 8
TOPK = 2
RBM = 256
BM = 256
BN1 = 2048
BN2 = 1024


def _gelu(v):
    return 0.5 * v * (1.0 + lax.erf(v * 0.7071067811865476))


def _router_body(x_ref, wr_ref, br_ref, o_ref):
    logits = jnp.dot(x_ref[...], wr_ref[...], preferred_element_type=jnp.float32)
    logits = logits + br_ref[...]
    ncol = logits.shape[1]
    col = lax.broadcasted_iota(jnp.int32, logits.shape, 1)
    m0 = jnp.max(logits, axis=1, keepdims=True)
    i0 = jnp.min(jnp.where(logits == m0, col, ncol), axis=1, keepdims=True)
    l2 = jnp.where(col == i0, -jnp.float32(jnp.inf), logits)
    m1 = jnp.max(l2, axis=1, keepdims=True)
    i1 = jnp.min(jnp.where(l2 == m1, col, ncol), axis=1, keepdims=True)
    o_ref[...] = jnp.where(col == 0, i0, jnp.where(col == 1, i1, 0)).astype(jnp.int32)


def _ffn1_body(be_ref, xs_ref, w1_ref, b1_ref, h_ref):
    acc = jnp.dot(xs_ref[...], w1_ref[0], preferred_element_type=jnp.float32)
    h_ref[...] = _gelu(acc + b1_ref[0])


def _ffn2_body(be_ref, h_ref, w2_ref, b2_ref, o_ref):
    acc = jnp.dot(h_ref[...], w2_ref[0], preferred_element_type=jnp.float32)
    o_ref[...] = acc + b2_ref[0]


def kernel(x, Wr, br, W1, b1, W2, b2):
    B, N, EMB = x.shape
    NE, _, HID = W1.shape
    T = B * N
    P = TOPK * T + NE * BM
    num_m = P // BM
    x_flat = x.reshape(T, EMB)

    wr_pad = jnp.zeros((EMB, 128), Wr.dtype).at[:, :NE].set(Wr)
    br_pad = jnp.full((1, 128), -1e30, br.dtype).at[0, :NE].set(br)
    topk = pl.pallas_call(
        _router_body,
        grid=(T // RBM,),
        in_specs=[
            pl.BlockSpec((RBM, EMB), lambda i: (i, 0)),
            pl.BlockSpec((EMB, 128), lambda i: (0, 0)),
            pl.BlockSpec((1, 128), lambda i: (0, 0)),
        ],
        out_specs=pl.BlockSpec((RBM, 128), lambda i: (i, 0)),
        out_shape=jax.ShapeDtypeStruct((T, 128), jnp.int32),
    )(x_flat, wr_pad, br_pad)
    e0 = topk[:, 0]
    e1 = topk[:, 1]

    ar = jnp.arange(NE, dtype=jnp.int32)
    oh = ((e0[:, None] == ar) | (e1[:, None] == ar)).astype(jnp.int32)
    cum = jnp.cumsum(oh, axis=0)
    counts = cum[-1]
    size_pad = ((counts + BM - 1) // BM) * BM
    start_pad = jnp.concatenate(
        [jnp.zeros((1,), jnp.int32), jnp.cumsum(size_pad)[:-1].astype(jnp.int32)])
    rank = cum - oh
    posm = start_pad[None, :] + rank
    pos0 = jnp.take_along_axis(posm, e0[:, None], axis=1)[:, 0]
    pos1 = jnp.take_along_axis(posm, e1[:, None], axis=1)[:, 0]
    tok = jnp.arange(T, dtype=jnp.int32)
    row_ids = jnp.zeros((P,), jnp.int32).at[pos0].set(tok).at[pos1].set(tok)
    blk_starts = jnp.arange(num_m, dtype=jnp.int32) * BM
    block_expert = jnp.clip(
        jnp.searchsorted(start_pad, blk_starts, side="right").astype(jnp.int32) - 1,
        0, NE - 1)

    xs = x_flat[row_ids]

    h = pl.pallas_call(
        _ffn1_body,
        grid_spec=pltpu.PrefetchScalarGridSpec(
            num_scalar_prefetch=1,
            grid=(HID // BN1, num_m),
            in_specs=[
                pl.BlockSpec((BM, EMB), lambda n, m, be: (m, 0)),
                pl.BlockSpec((1, EMB, BN1), lambda n, m, be: (be[m], 0, n)),
                pl.BlockSpec((1, 1, BN1), lambda n, m, be: (be[m], 0, n)),
            ],
            out_specs=pl.BlockSpec((BM, BN1), lambda n, m, be: (m, n)),
        ),
        out_shape=jax.ShapeDtypeStruct((P, HID), jnp.float32),
    )(block_expert, xs, W1, b1.reshape(NE, 1, HID))

    o = pl.pallas_call(
        _ffn2_body,
        grid_spec=pltpu.PrefetchScalarGridSpec(
            num_scalar_prefetch=1,
            grid=(EMB // BN2, num_m),
            in_specs=[
                pl.BlockSpec((BM, HID), lambda n, m, be: (m, 0)),
                pl.BlockSpec((1, HID, BN2), lambda n, m, be: (be[m], 0, n)),
                pl.BlockSpec((1, 1, BN2), lambda n, m, be: (be[m], 0, n)),
            ],
            out_specs=pl.BlockSpec((BM, BN2), lambda n, m, be: (m, n)),
        ),
        out_shape=jax.ShapeDtypeStruct((P, EMB), jnp.float32),
    )(block_expert, h, W2, b2.reshape(NE, 1, EMB))

    out = (o[pos0] + o[pos1]) * 0.5
    return out.reshape(B, N, EMB)
